# baseline (speedup 1.0000x reference)
import functools

import jax
import jax.numpy as jnp
from jax import lax
from jax.experimental import pallas as pl
from jax.experimental.pallas import tpu as pltpu
from jax.experimental.pallas import tpu_sc as plsc

N = 10000
B = 64
E = 320000
EPS = 1e-5
NC = 2
NS = 16
ROWS_PER_TILE = 624
TAIL0 = ROWS_PER_TILE * NS
TAIL = N - TAIL0
EPT = E // NS
CFG128 = (100, 10, 20, 3)
CFG64 = (125, 32, 5, 4)
BLK = 2000
NBLK = 2 * N // BLK
HIGHEST = None


def _make_edge_agg(D, cfg, tc_tiling=True):
  chunk, idxg, ngroup, nbuf = cfg
  assert chunk * idxg * ngroup == EPT
  mesh = plsc.VectorSubcoreMesh(core_axis_name="c", subcore_axis_name="s")
  params = pltpu.CompilerParams(use_tc_tiling_on_sc=tc_tiling)

  @functools.partial(
      pl.kernel,
      mesh=mesh,
      compiler_params=params,
      out_type=jax.ShapeDtypeStruct((2 * N, D), jnp.float32),
      scratch_types=(
          [pltpu.VMEM((2, idxg, chunk), jnp.int32),
           pltpu.VMEM((2, idxg, chunk), jnp.int32)]
          + [pltpu.VMEM((chunk, D), jnp.float32) for _ in range(nbuf)]
          + [pltpu.VMEM_SHARED((N, D), jnp.float32)]
          + [pltpu.SemaphoreType.DMA for _ in range(2 * nbuf + 1)]
      ),
  )
  def edge_agg(x_hbm, src_hbm, dst_hbm, zeros_hbm, out_hbm, src_v, dst_v,
               *rest):
    bufs = rest[:nbuf]
    acc = rest[nbuf]
    gsems = rest[nbuf + 1:2 * nbuf + 1]
    ssems = rest[2 * nbuf + 1:3 * nbuf + 1]
    isem = rest[3 * nbuf + 1]
    c = lax.axis_index("c")
    s = lax.axis_index("s")
    r0 = s * ROWS_PER_TILE
    pltpu.sync_copy(zeros_hbm, acc.at[pl.ds(r0, ROWS_PER_TILE)])

    @pl.when(s == NS - 1)
    def _():
      pltpu.sync_copy(zeros_hbm.at[pl.ds(0, TAIL)], acc.at[pl.ds(TAIL0, TAIL)])

    plsc.subcore_barrier()

    pltpu.async_copy(src_hbm.at[c, s, 0], src_v.at[0], isem)
    pltpu.async_copy(dst_hbm.at[c, s, 0], dst_v.at[0], isem)

    def group(g, carry):
      p = g % 2
      pltpu.make_async_copy(src_hbm.at[c, s, g], src_v.at[p], isem).wait()
      pltpu.make_async_copy(dst_hbm.at[c, s, g], dst_v.at[p], isem).wait()

      @pl.when(g + 1 < ngroup)
      def _():
        pltpu.async_copy(src_hbm.at[c, s, g + 1], src_v.at[1 - p], isem)
        pltpu.async_copy(dst_hbm.at[c, s, g + 1], dst_v.at[1 - p], isem)

      gd = [None] * idxg
      sd = [None] * idxg
      gd[0] = pltpu.async_copy(x_hbm.at[src_v.at[p, 0]], bufs[0], gsems[0])
      for j in range(idxg):
        b = j % nbuf
        gd[j].wait()
        if j + 1 < idxg:
          if j + 1 - nbuf >= 0:
            sd[j + 1 - nbuf].wait()
          gd[j + 1] = pltpu.async_copy(x_hbm.at[src_v.at[p, j + 1]],
                                       bufs[(j + 1) % nbuf],
                                       gsems[(j + 1) % nbuf])
        sd[j] = pltpu.async_copy(bufs[b], acc.at[dst_v.at[p, j]], ssems[b],
                                 add=True)
      for t in range(max(idxg - nbuf, 0), idxg):
        sd[t].wait()
      return carry

    lax.fori_loop(0, ngroup, group, 0)
    plsc.subcore_barrier()
    pltpu.sync_copy(acc.at[pl.ds(r0, ROWS_PER_TILE)],
                    out_hbm.at[pl.ds(c * N + r0, ROWS_PER_TILE)])

    @pl.when(s == NS - 1)
    def _():
      pltpu.sync_copy(acc.at[pl.ds(TAIL0, TAIL)],
                      out_hbm.at[pl.ds(c * N + TAIL0, TAIL)])

  return edge_agg


_edge_agg_128 = _make_edge_agg(128, CFG128)
_edge_agg_64 = _make_edge_agg(64, CFG64, tc_tiling=False)


def _onehot(bat_ref):
  iot = lax.broadcasted_iota(jnp.int32, (1, 2 * B), 1)
  return (bat_ref[:] == iot).astype(jnp.float32)


def _segdot(pm, x):
  x1 = x.astype(jnp.bfloat16).astype(jnp.float32)
  r1 = x - x1
  x2 = r1.astype(jnp.bfloat16).astype(jnp.float32)
  x3 = r1 - x2

  def d(t):
    return lax.dot_general(pm, t, (((0,), (0,)), ((), ())),
                           preferred_element_type=jnp.float32)

  return d(x1) + d(x2) + d(x3)


def _bcast(pm, v):
  m = jnp.broadcast_to(v[:, None], (2 * B, 64))
  m1 = m.astype(jnp.bfloat16).astype(jnp.float32)
  m2 = m - m1

  def d(t):
    return jnp.dot(pm, t, preferred_element_type=jnp.float32)

  return d(m1) + d(m2)


def _mlp_stats(h_in, bat_ref, w1, b1, w2, b2, h_ref, s1_ref, deg_ref,
               with_deg):
  t = jnp.maximum(
      jnp.dot(h_in, w1, preferred_element_type=jnp.float32,
              precision=HIGHEST) + b1, 0.0)
  hv = jnp.dot(t, w2, preferred_element_type=jnp.float32,
               precision=HIGHEST) + b2
  h_ref[:] = hv
  pm = _onehot(bat_ref)

  @pl.when(pl.program_id(0) == 0)
  def _():
    s1_ref[:] = jnp.zeros_like(s1_ref)
    if with_deg:
      deg_ref[:] = jnp.zeros_like(deg_ref)

  s1_ref[:] += _segdot(pm, hv)
  if with_deg:
    deg_ref[:] += jnp.sum(pm, axis=0, keepdims=True)


def _mlp1_stats(xs_ref, agg_ref, bat_ref, w1_ref, b1_ref, w2_ref, b2_ref,
                h_ref, s1_ref, deg_ref):
  h_in = xs_ref[:] + agg_ref[:]
  _mlp_stats(h_in, bat_ref, w1_ref[:], b1_ref[:], w2_ref[:], b2_ref[:],
             h_ref, s1_ref, deg_ref, True)


def _mlp2_stats(h_ref, agg_ref, bat_ref, w1_ref, b1_ref, w2_ref, b2_ref,
                h2_ref, s1_ref):
  h_in = h_ref[:] + agg_ref[:]
  _mlp_stats(h_in, bat_ref, w1_ref[:], b1_ref[:], w2_ref[:], b2_ref[:],
             h2_ref, s1_ref, None, False)


def _seg_mean(s1, deg):
  norm = jnp.maximum(deg, 1.0) * 64.0
  return jnp.sum(s1, axis=1) / norm, norm


def _var_pass(h_ref, bat_ref, s1_ref, deg_ref, segv_ref):
  mean, _ = _seg_mean(s1_ref[:], deg_ref[0, :])
  pm = _onehot(bat_ref)
  xc = h_ref[:] - _bcast(pm, mean)

  @pl.when(pl.program_id(0) == 0)
  def _():
    segv_ref[:] = jnp.zeros_like(segv_ref)

  segv_ref[:] += _segdot(pm, xc * xc)


def _ln_fields(h, bat_ref, s1, segv, deg):
  mean, norm = _seg_mean(s1, deg)
  var = jnp.sum(segv, axis=1) / norm
  scale = lax.rsqrt(var + EPS)
  pm = _onehot(bat_ref)
  xc = h - _bcast(pm, mean)
  return pm, xc * _bcast(pm, scale)


def _ln1_apply(h_ref, bat_ref, s1_ref, segv_ref, deg_ref, lnw_ref, lnb_ref,
               out_ref):
  _, xn = _ln_fields(h_ref[:], bat_ref, s1_ref[:], segv_ref[:], deg_ref[0, :])
  out_ref[:] = jnp.maximum(xn * lnw_ref[:] + lnb_ref[:], 0.0)


def _ln2_pool(h_ref, bat_ref, s1_ref, segv_ref, deg_ref, lnw_ref, lnb_ref,
              pool_ref):
  pm, xn = _ln_fields(h_ref[:], bat_ref, s1_ref[:], segv_ref[:],
                      deg_ref[0, :])
  res = jnp.maximum(xn * lnw_ref[:] + lnb_ref[:], 0.0)

  @pl.when(pl.program_id(0) == 0)
  def _():
    pool_ref[:] = jnp.zeros_like(pool_ref)

  pool_ref[:] += _segdot(pm, res)


def _head(s_ref, deg_ref, d1_ref, d2_ref, fwa_ref, fwb_ref, fwc_ref,
          fwd_ref, f1b_ref, f2w_ref, f2b_ref, ow_ref, ob_ref, out_ref):
  cnt = jnp.maximum(deg_ref[0, :], 1.0)
  s = s_ref[:]
  emb = s + s / cnt[:, None]
  e1 = emb[0:B]
  e2 = emb[B:2 * B]
  hh = (jnp.dot(e1, fwa_ref[:], preferred_element_type=jnp.float32,
                precision=HIGHEST)
        + jnp.dot(e2, fwb_ref[:], preferred_element_type=jnp.float32,
                  precision=HIGHEST)
        + jnp.dot(d1_ref[:], fwc_ref[:], preferred_element_type=jnp.float32,
                  precision=HIGHEST)
        + jnp.dot(d2_ref[:], fwd_ref[:], preferred_element_type=jnp.float32,
                  precision=HIGHEST)
        + f1b_ref[:])
  hh = jnp.maximum(hh, 0.0)
  hh = jnp.maximum(
      jnp.dot(hh, f2w_ref[:], preferred_element_type=jnp.float32,
              precision=HIGHEST) + f2b_ref[:], 0.0)
  out_ref[:] = (jnp.dot(hh, ow_ref[:], preferred_element_type=jnp.float32,
                        precision=HIGHEST) + ob_ref[:])


def _row_spec(width):
  return pl.BlockSpec((BLK, width), lambda i: (i, 0))


def _fix_spec(shape):
  return pl.BlockSpec(shape, lambda i: (0, 0))


def kernel(g1_x, g1_edge_index, g1_batch, g2_x, g2_edge_index, g2_batch,
           d1, d2,
           nn1_w1, nn1_b1, nn1_w2, nn1_b2, ln1_w, ln1_b,
           nn2_w1, nn2_b1, nn2_w2, nn2_b2, ln2_w, ln2_b,
           fc1_w, fc1_b, fc2_w, fc2_b, out_w, out_b):
  f32 = jnp.float32
  xs = jnp.concatenate([g1_x, g2_x], axis=0)
  srcf = jnp.stack([g1_edge_index[0], g2_edge_index[0] + N]).astype(jnp.int32)
  dstf = jnp.stack([g1_edge_index[1], g2_edge_index[1]]).astype(jnp.int32)

  def shaped(cfg):
    chunk, idxg, ngroup, _ = cfg
    shape = (NC, NS, ngroup, idxg, chunk)
    return srcf.reshape(shape), dstf.reshape(shape)

  src, dst = shaped(CFG128)
  src64, dst64 = shaped(CFG64)
  z128 = jnp.zeros((ROWS_PER_TILE, 128), f32)
  z64 = jnp.zeros((ROWS_PER_TILE, 64), f32)
  bat = jnp.concatenate([g1_batch, g2_batch + B]).astype(jnp.int32)
  bat = bat.reshape(2 * N, 1)

  agg1 = _edge_agg_128(xs, src, dst, z128)

  hm, s1a, deg = pl.pallas_call(
      _mlp1_stats,
      grid=(NBLK,),
      in_specs=[
          _row_spec(128), _row_spec(128), _row_spec(1),
          _fix_spec((128, 64)), _fix_spec((1, 64)),
          _fix_spec((64, 64)), _fix_spec((1, 64)),
      ],
      out_specs=[
          _row_spec(64),
          _fix_spec((2 * B, 64)),
          _fix_spec((1, 2 * B)),
      ],
      out_shape=[
          jax.ShapeDtypeStruct((2 * N, 64), f32),
          jax.ShapeDtypeStruct((2 * B, 64), f32),
          jax.ShapeDtypeStruct((1, 2 * B), f32),
      ],
  )(xs, agg1, bat, nn1_w1, nn1_b1.reshape(1, -1), nn1_w2,
    nn1_b2.reshape(1, -1))

  var_specs = dict(
      grid=(NBLK,),
      in_specs=[
          _row_spec(64), _row_spec(1),
          _fix_spec((2 * B, 64)), _fix_spec((1, 2 * B)),
      ],
      out_specs=_fix_spec((2 * B, 64)),
      out_shape=jax.ShapeDtypeStruct((2 * B, 64), f32),
  )

  segva = pl.pallas_call(_var_pass, **var_specs)(hm, bat, s1a, deg)

  h = pl.pallas_call(
      _ln1_apply,
      grid=(NBLK,),
      in_specs=[
          _row_spec(64), _row_spec(1),
          _fix_spec((2 * B, 64)), _fix_spec((2 * B, 64)),
          _fix_spec((1, 2 * B)), _fix_spec((1, 64)), _fix_spec((1, 64)),
      ],
      out_specs=_row_spec(64),
      out_shape=jax.ShapeDtypeStruct((2 * N, 64), f32),
  )(hm, bat, s1a, segva, deg, ln1_w.reshape(1, -1), ln1_b.reshape(1, -1))

  agg2 = _edge_agg_64(h, src64, dst64, z64)

  h2m, s1b = pl.pallas_call(
      _mlp2_stats,
      grid=(NBLK,),
      in_specs=[
          _row_spec(64), _row_spec(64), _row_spec(1),
          _fix_spec((64, 64)), _fix_spec((1, 64)),
          _fix_spec((64, 64)), _fix_spec((1, 64)),
      ],
      out_specs=[
          _row_spec(64),
          _fix_spec((2 * B, 64)),
      ],
      out_shape=[
          jax.ShapeDtypeStruct((2 * N, 64), f32),
          jax.ShapeDtypeStruct((2 * B, 64), f32),
      ],
  )(h, agg2, bat, nn2_w1, nn2_b1.reshape(1, -1), nn2_w2,
    nn2_b2.reshape(1, -1))

  segvb = pl.pallas_call(_var_pass, **var_specs)(h2m, bat, s1b, deg)

  pool = pl.pallas_call(
      _ln2_pool,
      grid=(NBLK,),
      in_specs=[
          _row_spec(64), _row_spec(1),
          _fix_spec((2 * B, 64)), _fix_spec((2 * B, 64)),
          _fix_spec((1, 2 * B)), _fix_spec((1, 64)), _fix_spec((1, 64)),
      ],
      out_specs=_fix_spec((2 * B, 64)),
      out_shape=jax.ShapeDtypeStruct((2 * B, 64), f32),
  )(h2m, bat, s1b, segvb, deg, ln2_w.reshape(1, -1), ln2_b.reshape(1, -1))

  out = pl.pallas_call(
      _head,
      out_shape=jax.ShapeDtypeStruct((B, 1), f32),
  )(pool, deg, d1, d2, fc1_w[0:B], fc1_w[B:2 * B], fc1_w[2 * B:2 * B + 5],
    fc1_w[2 * B + 5:2 * B + 10], fc1_b.reshape(1, -1), fc2_w,
    fc2_b.reshape(1, -1), out_w, out_b.reshape(1, -1))
  return out


# speedup vs baseline: 1.0390x; 1.0390x over previous
import functools

import jax
import jax.numpy as jnp
from jax import lax
from jax.experimental import pallas as pl
from jax.experimental.pallas import tpu as pltpu
from jax.experimental.pallas import tpu_sc as plsc

N = 10000
B = 64
E = 320000
EPS = 1e-5
NC = 2
NS = 16
ROWS_PER_TILE = 624
TAIL0 = ROWS_PER_TILE * NS
TAIL = N - TAIL0
EPT = E // NS
CFG128 = (125, 16, 10, 2)
CFG64 = (125, 32, 5, 4)
BLK = 2000
NBLK = 2 * N // BLK
HIGHEST = None


def _make_edge_agg(D, cfg, tc_tiling=True):
  chunk, idxg, ngroup, nbuf = cfg
  assert chunk * idxg * ngroup == EPT
  mesh = plsc.VectorSubcoreMesh(core_axis_name="c", subcore_axis_name="s")
  params = pltpu.CompilerParams(use_tc_tiling_on_sc=tc_tiling)

  @functools.partial(
      pl.kernel,
      mesh=mesh,
      compiler_params=params,
      out_type=jax.ShapeDtypeStruct((2 * N, D), jnp.float32),
      scratch_types=(
          [pltpu.VMEM((2, idxg, chunk), jnp.int32),
           pltpu.VMEM((2, idxg, chunk), jnp.int32)]
          + [pltpu.VMEM((chunk, D), jnp.float32) for _ in range(nbuf)]
          + [pltpu.VMEM_SHARED((N, D), jnp.float32)]
          + [pltpu.SemaphoreType.DMA for _ in range(2 * nbuf + 1)]
      ),
  )
  def edge_agg(x_hbm, src_hbm, dst_hbm, zeros_hbm, out_hbm, src_v, dst_v,
               *rest):
    bufs = rest[:nbuf]
    acc = rest[nbuf]
    gsems = rest[nbuf + 1:2 * nbuf + 1]
    ssems = rest[2 * nbuf + 1:3 * nbuf + 1]
    isem = rest[3 * nbuf + 1]
    c = lax.axis_index("c")
    s = lax.axis_index("s")
    r0 = s * ROWS_PER_TILE
    pltpu.sync_copy(zeros_hbm, acc.at[pl.ds(r0, ROWS_PER_TILE)])

    @pl.when(s == NS - 1)
    def _():
      pltpu.sync_copy(zeros_hbm.at[pl.ds(0, TAIL)], acc.at[pl.ds(TAIL0, TAIL)])

    plsc.subcore_barrier()

    pltpu.async_copy(src_hbm.at[c, s, 0], src_v.at[0], isem)
    pltpu.async_copy(dst_hbm.at[c, s, 0], dst_v.at[0], isem)

    def group(g, carry):
      p = g % 2
      pltpu.make_async_copy(src_hbm.at[c, s, g], src_v.at[p], isem).wait()
      pltpu.make_async_copy(dst_hbm.at[c, s, g], dst_v.at[p], isem).wait()

      @pl.when(g + 1 < ngroup)
      def _():
        pltpu.async_copy(src_hbm.at[c, s, g + 1], src_v.at[1 - p], isem)
        pltpu.async_copy(dst_hbm.at[c, s, g + 1], dst_v.at[1 - p], isem)

      gd = [None] * idxg
      sd = [None] * idxg
      gd[0] = pltpu.async_copy(x_hbm.at[src_v.at[p, 0]], bufs[0], gsems[0])
      for j in range(idxg):
        b = j % nbuf
        gd[j].wait()
        if j + 1 < idxg:
          if j + 1 - nbuf >= 0:
            sd[j + 1 - nbuf].wait()
          gd[j + 1] = pltpu.async_copy(x_hbm.at[src_v.at[p, j + 1]],
                                       bufs[(j + 1) % nbuf],
                                       gsems[(j + 1) % nbuf])
        sd[j] = pltpu.async_copy(bufs[b], acc.at[dst_v.at[p, j]], ssems[b],
                                 add=True)
      for t in range(max(idxg - nbuf, 0), idxg):
        sd[t].wait()
      return carry

    lax.fori_loop(0, ngroup, group, 0)
    plsc.subcore_barrier()
    pltpu.sync_copy(acc.at[pl.ds(r0, ROWS_PER_TILE)],
                    out_hbm.at[pl.ds(c * N + r0, ROWS_PER_TILE)])

    @pl.when(s == NS - 1)
    def _():
      pltpu.sync_copy(acc.at[pl.ds(TAIL0, TAIL)],
                      out_hbm.at[pl.ds(c * N + TAIL0, TAIL)])

  return edge_agg


_edge_agg_128 = _make_edge_agg(128, CFG128)
_edge_agg_64 = _make_edge_agg(64, CFG64, tc_tiling=False)


def _onehot(bat_ref):
  iot = lax.broadcasted_iota(jnp.int32, (1, 2 * B), 1)
  return (bat_ref[:] == iot).astype(jnp.float32)


def _segdot(pm, x):
  x1 = x.astype(jnp.bfloat16).astype(jnp.float32)
  r1 = x - x1
  x2 = r1.astype(jnp.bfloat16).astype(jnp.float32)
  x3 = r1 - x2

  def d(t):
    return lax.dot_general(pm, t, (((0,), (0,)), ((), ())),
                           preferred_element_type=jnp.float32)

  return d(x1) + d(x2) + d(x3)


def _bcast(pm, v):
  m = jnp.broadcast_to(v[:, None], (2 * B, 64))
  m1 = m.astype(jnp.bfloat16).astype(jnp.float32)
  m2 = m - m1

  def d(t):
    return jnp.dot(pm, t, preferred_element_type=jnp.float32)

  return d(m1) + d(m2)


def _mlp_stats(h_in, bat_ref, w1, b1, w2, b2, h_ref, s1_ref, deg_ref,
               with_deg):
  t = jnp.maximum(
      jnp.dot(h_in, w1, preferred_element_type=jnp.float32,
              precision=HIGHEST) + b1, 0.0)
  hv = jnp.dot(t, w2, preferred_element_type=jnp.float32,
               precision=HIGHEST) + b2
  h_ref[:] = hv
  pm = _onehot(bat_ref)

  @pl.when(pl.program_id(0) == 0)
  def _():
    s1_ref[:] = jnp.zeros_like(s1_ref)
    if with_deg:
      deg_ref[:] = jnp.zeros_like(deg_ref)

  s1_ref[:] += _segdot(pm, hv)
  if with_deg:
    deg_ref[:] += jnp.sum(pm, axis=0, keepdims=True)


def _mlp1_stats(xs_ref, agg_ref, bat_ref, w1_ref, b1_ref, w2_ref, b2_ref,
                h_ref, s1_ref, deg_ref):
  h_in = xs_ref[:] + agg_ref[:]
  _mlp_stats(h_in, bat_ref, w1_ref[:], b1_ref[:], w2_ref[:], b2_ref[:],
             h_ref, s1_ref, deg_ref, True)


def _mlp2_stats(h_ref, agg_ref, bat_ref, w1_ref, b1_ref, w2_ref, b2_ref,
                h2_ref, s1_ref):
  h_in = h_ref[:] + agg_ref[:]
  _mlp_stats(h_in, bat_ref, w1_ref[:], b1_ref[:], w2_ref[:], b2_ref[:],
             h2_ref, s1_ref, None, False)


def _seg_mean(s1, deg):
  norm = jnp.maximum(deg, 1.0) * 64.0
  return jnp.sum(s1, axis=1) / norm, norm


def _var_pass(h_ref, bat_ref, s1_ref, deg_ref, segv_ref):
  mean, _ = _seg_mean(s1_ref[:], deg_ref[0, :])
  pm = _onehot(bat_ref)
  xc = h_ref[:] - _bcast(pm, mean)

  @pl.when(pl.program_id(0) == 0)
  def _():
    segv_ref[:] = jnp.zeros_like(segv_ref)

  segv_ref[:] += _segdot(pm, xc * xc)


def _ln_fields(h, bat_ref, s1, segv, deg):
  mean, norm = _seg_mean(s1, deg)
  var = jnp.sum(segv, axis=1) / norm
  scale = lax.rsqrt(var + EPS)
  pm = _onehot(bat_ref)
  xc = h - _bcast(pm, mean)
  return pm, xc * _bcast(pm, scale)


def _ln1_apply(h_ref, bat_ref, s1_ref, segv_ref, deg_ref, lnw_ref, lnb_ref,
               out_ref):
  _, xn = _ln_fields(h_ref[:], bat_ref, s1_ref[:], segv_ref[:], deg_ref[0, :])
  out_ref[:] = jnp.maximum(xn * lnw_ref[:] + lnb_ref[:], 0.0)


def _ln2_pool(h_ref, bat_ref, s1_ref, segv_ref, deg_ref, lnw_ref, lnb_ref,
              pool_ref):
  pm, xn = _ln_fields(h_ref[:], bat_ref, s1_ref[:], segv_ref[:],
                      deg_ref[0, :])
  res = jnp.maximum(xn * lnw_ref[:] + lnb_ref[:], 0.0)

  @pl.when(pl.program_id(0) == 0)
  def _():
    pool_ref[:] = jnp.zeros_like(pool_ref)

  pool_ref[:] += _segdot(pm, res)


def _head(s_ref, deg_ref, d1_ref, d2_ref, fwa_ref, fwb_ref, fwc_ref,
          fwd_ref, f1b_ref, f2w_ref, f2b_ref, ow_ref, ob_ref, out_ref):
  cnt = jnp.maximum(deg_ref[0, :], 1.0)
  s = s_ref[:]
  emb = s + s / cnt[:, None]
  e1 = emb[0:B]
  e2 = emb[B:2 * B]
  hh = (jnp.dot(e1, fwa_ref[:], preferred_element_type=jnp.float32,
                precision=HIGHEST)
        + jnp.dot(e2, fwb_ref[:], preferred_element_type=jnp.float32,
                  precision=HIGHEST)
        + jnp.dot(d1_ref[:], fwc_ref[:], preferred_element_type=jnp.float32,
                  precision=HIGHEST)
        + jnp.dot(d2_ref[:], fwd_ref[:], preferred_element_type=jnp.float32,
                  precision=HIGHEST)
        + f1b_ref[:])
  hh = jnp.maximum(hh, 0.0)
  hh = jnp.maximum(
      jnp.dot(hh, f2w_ref[:], preferred_element_type=jnp.float32,
              precision=HIGHEST) + f2b_ref[:], 0.0)
  out_ref[:] = (jnp.dot(hh, ow_ref[:], preferred_element_type=jnp.float32,
                        precision=HIGHEST) + ob_ref[:])


def _row_spec(width):
  return pl.BlockSpec((BLK, width), lambda i: (i, 0))


def _fix_spec(shape):
  return pl.BlockSpec(shape, lambda i: (0, 0))


def kernel(g1_x, g1_edge_index, g1_batch, g2_x, g2_edge_index, g2_batch,
           d1, d2,
           nn1_w1, nn1_b1, nn1_w2, nn1_b2, ln1_w, ln1_b,
           nn2_w1, nn2_b1, nn2_w2, nn2_b2, ln2_w, ln2_b,
           fc1_w, fc1_b, fc2_w, fc2_b, out_w, out_b):
  f32 = jnp.float32
  xs = jnp.concatenate([g1_x, g2_x], axis=0)
  srcf = jnp.stack([g1_edge_index[0], g2_edge_index[0] + N]).astype(jnp.int32)
  dstf = jnp.stack([g1_edge_index[1], g2_edge_index[1]]).astype(jnp.int32)

  def shaped(cfg):
    chunk, idxg, ngroup, _ = cfg
    shape = (NC, NS, ngroup, idxg, chunk)
    return srcf.reshape(shape), dstf.reshape(shape)

  src, dst = shaped(CFG128)
  src64, dst64 = shaped(CFG64)
  z128 = jnp.zeros((ROWS_PER_TILE, 128), f32)
  z64 = jnp.zeros((ROWS_PER_TILE, 64), f32)
  bat = jnp.concatenate([g1_batch, g2_batch + B]).astype(jnp.int32)
  bat = bat.reshape(2 * N, 1)

  agg1 = _edge_agg_128(xs, src, dst, z128)

  hm, s1a, deg = pl.pallas_call(
      _mlp1_stats,
      grid=(NBLK,),
      in_specs=[
          _row_spec(128), _row_spec(128), _row_spec(1),
          _fix_spec((128, 64)), _fix_spec((1, 64)),
          _fix_spec((64, 64)), _fix_spec((1, 64)),
      ],
      out_specs=[
          _row_spec(64),
          _fix_spec((2 * B, 64)),
          _fix_spec((1, 2 * B)),
      ],
      out_shape=[
          jax.ShapeDtypeStruct((2 * N, 64), f32),
          jax.ShapeDtypeStruct((2 * B, 64), f32),
          jax.ShapeDtypeStruct((1, 2 * B), f32),
      ],
  )(xs, agg1, bat, nn1_w1, nn1_b1.reshape(1, -1), nn1_w2,
    nn1_b2.reshape(1, -1))

  var_specs = dict(
      grid=(NBLK,),
      in_specs=[
          _row_spec(64), _row_spec(1),
          _fix_spec((2 * B, 64)), _fix_spec((1, 2 * B)),
      ],
      out_specs=_fix_spec((2 * B, 64)),
      out_shape=jax.ShapeDtypeStruct((2 * B, 64), f32),
  )

  segva = pl.pallas_call(_var_pass, **var_specs)(hm, bat, s1a, deg)

  h = pl.pallas_call(
      _ln1_apply,
      grid=(NBLK,),
      in_specs=[
          _row_spec(64), _row_spec(1),
          _fix_spec((2 * B, 64)), _fix_spec((2 * B, 64)),
          _fix_spec((1, 2 * B)), _fix_spec((1, 64)), _fix_spec((1, 64)),
      ],
      out_specs=_row_spec(64),
      out_shape=jax.ShapeDtypeStruct((2 * N, 64), f32),
  )(hm, bat, s1a, segva, deg, ln1_w.reshape(1, -1), ln1_b.reshape(1, -1))

  agg2 = _edge_agg_64(h, src64, dst64, z64)

  h2m, s1b = pl.pallas_call(
      _mlp2_stats,
      grid=(NBLK,),
      in_specs=[
          _row_spec(64), _row_spec(64), _row_spec(1),
          _fix_spec((64, 64)), _fix_spec((1, 64)),
          _fix_spec((64, 64)), _fix_spec((1, 64)),
      ],
      out_specs=[
          _row_spec(64),
          _fix_spec((2 * B, 64)),
      ],
      out_shape=[
          jax.ShapeDtypeStruct((2 * N, 64), f32),
          jax.ShapeDtypeStruct((2 * B, 64), f32),
      ],
  )(h, agg2, bat, nn2_w1, nn2_b1.reshape(1, -1), nn2_w2,
    nn2_b2.reshape(1, -1))

  segvb = pl.pallas_call(_var_pass, **var_specs)(h2m, bat, s1b, deg)

  pool = pl.pallas_call(
      _ln2_pool,
      grid=(NBLK,),
      in_specs=[
          _row_spec(64), _row_spec(1),
          _fix_spec((2 * B, 64)), _fix_spec((2 * B, 64)),
          _fix_spec((1, 2 * B)), _fix_spec((1, 64)), _fix_spec((1, 64)),
      ],
      out_specs=_fix_spec((2 * B, 64)),
      out_shape=jax.ShapeDtypeStruct((2 * B, 64), f32),
  )(h2m, bat, s1b, segvb, deg, ln2_w.reshape(1, -1), ln2_b.reshape(1, -1))

  out = pl.pallas_call(
      _head,
      out_shape=jax.ShapeDtypeStruct((B, 1), f32),
  )(pool, deg, d1, d2, fc1_w[0:B], fc1_w[B:2 * B], fc1_w[2 * B:2 * B + 5],
    fc1_w[2 * B + 5:2 * B + 10], fc1_b.reshape(1, -1), fc2_w,
    fc2_b.reshape(1, -1), out_w, out_b.reshape(1, -1))
  return out


# speedup vs baseline: 1.1046x; 1.0632x over previous
import functools

import jax
import jax.numpy as jnp
from jax import lax
from jax.experimental import pallas as pl
from jax.experimental.pallas import tpu as pltpu
from jax.experimental.pallas import tpu_sc as plsc

N = 10000
B = 64
E = 320000
EPS = 1e-5
NC = 2
NS = 16
ROWS_PER_TILE = 624
TAIL0 = ROWS_PER_TILE * NS
TAIL = N - TAIL0
EPT = E // NS
CFG128 = (125, 16, 10, 2)
CFG64 = (125, 32, 5, 4)
BLK = 2000
NBLK = 2 * N // BLK
HIGHEST = None


def _make_edge_agg(D, cfg, tc_tiling=True):
  chunk, idxg, ngroup, nbuf = cfg
  assert chunk * idxg * ngroup == EPT
  mesh = plsc.VectorSubcoreMesh(core_axis_name="c", subcore_axis_name="s")
  params = pltpu.CompilerParams(use_tc_tiling_on_sc=tc_tiling)

  @functools.partial(
      pl.kernel,
      mesh=mesh,
      compiler_params=params,
      out_type=jax.ShapeDtypeStruct((2 * N, D), jnp.float32),
      scratch_types=(
          [pltpu.VMEM((2, idxg, chunk), jnp.int32),
           pltpu.VMEM((2, idxg, chunk), jnp.int32)]
          + [pltpu.VMEM((chunk, D), jnp.float32) for _ in range(nbuf)]
          + [pltpu.VMEM_SHARED((N, D), jnp.float32)]
          + [pltpu.SemaphoreType.DMA for _ in range(2 * nbuf + 1)]
      ),
  )
  def edge_agg(x_hbm, src_hbm, dst_hbm, zeros_hbm, out_hbm, src_v, dst_v,
               *rest):
    bufs = rest[:nbuf]
    acc = rest[nbuf]
    gsems = rest[nbuf + 1:2 * nbuf + 1]
    ssems = rest[2 * nbuf + 1:3 * nbuf + 1]
    isem = rest[3 * nbuf + 1]
    c = lax.axis_index("c")
    s = lax.axis_index("s")
    r0 = s * ROWS_PER_TILE
    pltpu.sync_copy(zeros_hbm, acc.at[pl.ds(r0, ROWS_PER_TILE)])

    @pl.when(s == NS - 1)
    def _():
      pltpu.sync_copy(zeros_hbm.at[pl.ds(0, TAIL)], acc.at[pl.ds(TAIL0, TAIL)])

    plsc.subcore_barrier()

    pltpu.async_copy(src_hbm.at[c, s, 0], src_v.at[0], isem)
    pltpu.async_copy(dst_hbm.at[c, s, 0], dst_v.at[0], isem)

    def group(g, carry):
      p = g % 2
      pltpu.make_async_copy(src_hbm.at[c, s, g], src_v.at[p], isem).wait()
      pltpu.make_async_copy(dst_hbm.at[c, s, g], dst_v.at[p], isem).wait()

      @pl.when(g + 1 < ngroup)
      def _():
        pltpu.async_copy(src_hbm.at[c, s, g + 1], src_v.at[1 - p], isem)
        pltpu.async_copy(dst_hbm.at[c, s, g + 1], dst_v.at[1 - p], isem)

      gd = [None] * idxg
      sd = [None] * idxg
      gd[0] = pltpu.async_copy(x_hbm.at[src_v.at[p, 0]], bufs[0], gsems[0])
      for j in range(idxg):
        b = j % nbuf
        gd[j].wait()
        if j + 1 < idxg:
          if j + 1 - nbuf >= 0:
            sd[j + 1 - nbuf].wait()
          gd[j + 1] = pltpu.async_copy(x_hbm.at[src_v.at[p, j + 1]],
                                       bufs[(j + 1) % nbuf],
                                       gsems[(j + 1) % nbuf])
        sd[j] = pltpu.async_copy(bufs[b], acc.at[dst_v.at[p, j]], ssems[b],
                                 add=True)
      for t in range(max(idxg - nbuf, 0), idxg):
        sd[t].wait()
      return carry

    lax.fori_loop(0, ngroup, group, 0)
    plsc.subcore_barrier()
    pltpu.sync_copy(acc.at[pl.ds(r0, ROWS_PER_TILE)],
                    out_hbm.at[pl.ds(c * N + r0, ROWS_PER_TILE)])

    @pl.when(s == NS - 1)
    def _():
      pltpu.sync_copy(acc.at[pl.ds(TAIL0, TAIL)],
                      out_hbm.at[pl.ds(c * N + TAIL0, TAIL)])

  return edge_agg


_edge_agg_128 = _make_edge_agg(128, CFG128)
_edge_agg_64 = _make_edge_agg(64, CFG64, tc_tiling=False)


def _onehot(bat_ref):
  iot = lax.broadcasted_iota(jnp.int32, (1, 2 * B), 1)
  return (bat_ref[:] == iot).astype(jnp.float32)


def _segdot(pm, x):
  x1 = x.astype(jnp.bfloat16).astype(jnp.float32)
  r1 = x - x1
  x2 = r1.astype(jnp.bfloat16).astype(jnp.float32)
  x3 = r1 - x2

  def d(t):
    return lax.dot_general(pm, t, (((0,), (0,)), ((), ())),
                           preferred_element_type=jnp.float32)

  return d(x1) + d(x2) + d(x3)


def _bcast(pm, v):
  m = jnp.broadcast_to(v[:, None], (2 * B, 64))
  m1 = m.astype(jnp.bfloat16).astype(jnp.float32)
  m2 = m - m1

  def d(t):
    return jnp.dot(pm, t, preferred_element_type=jnp.float32)

  return d(m1) + d(m2)


def _mlp_stats(h_in, bat_ref, w1, b1, w2, b2, h_ref, s1_ref, s2_ref,
               deg_ref, with_deg):
  t = jnp.maximum(
      jnp.dot(h_in, w1, preferred_element_type=jnp.float32,
              precision=HIGHEST) + b1, 0.0)
  hv = jnp.dot(t, w2, preferred_element_type=jnp.float32,
               precision=HIGHEST) + b2
  h_ref[:] = hv
  pm = _onehot(bat_ref)

  @pl.when(pl.program_id(0) == 0)
  def _():
    s1_ref[:] = jnp.zeros_like(s1_ref)
    s2_ref[:] = jnp.zeros_like(s2_ref)
    if with_deg:
      deg_ref[:] = jnp.zeros_like(deg_ref)

  s1_ref[:] += _segdot(pm, hv)
  s2_ref[:] += _segdot(pm, hv * hv)
  if with_deg:
    deg_ref[:] += jnp.sum(pm, axis=0, keepdims=True)


def _mlp1_stats(xs_ref, agg_ref, bat_ref, w1_ref, b1_ref, w2_ref, b2_ref,
                h_ref, s1_ref, s2_ref, deg_ref):
  h_in = xs_ref[:] + agg_ref[:]
  _mlp_stats(h_in, bat_ref, w1_ref[:], b1_ref[:], w2_ref[:], b2_ref[:],
             h_ref, s1_ref, s2_ref, deg_ref, True)


def _mlp2_stats(h_ref, agg_ref, bat_ref, w1_ref, b1_ref, w2_ref, b2_ref,
                h2_ref, s1_ref, s2_ref):
  h_in = h_ref[:] + agg_ref[:]
  _mlp_stats(h_in, bat_ref, w1_ref[:], b1_ref[:], w2_ref[:], b2_ref[:],
             h2_ref, s1_ref, s2_ref, None, False)


def _ln_fields(h, bat_ref, s1, s2, deg):
  norm = jnp.maximum(deg, 1.0) * 64.0
  mean = jnp.sum(s1, axis=1) / norm
  var = jnp.sum(s2, axis=1) / norm - mean * mean
  scale = lax.rsqrt(var + EPS)
  pm = _onehot(bat_ref)
  xc = h - _bcast(pm, mean)
  return pm, xc * _bcast(pm, scale)


def _ln1_apply(h_ref, bat_ref, s1_ref, segv_ref, deg_ref, lnw_ref, lnb_ref,
               out_ref):
  _, xn = _ln_fields(h_ref[:], bat_ref, s1_ref[:], segv_ref[:], deg_ref[0, :])
  out_ref[:] = jnp.maximum(xn * lnw_ref[:] + lnb_ref[:], 0.0)


def _ln2_pool(h_ref, bat_ref, s1_ref, segv_ref, deg_ref, lnw_ref, lnb_ref,
              pool_ref):
  pm, xn = _ln_fields(h_ref[:], bat_ref, s1_ref[:], segv_ref[:],
                      deg_ref[0, :])
  res = jnp.maximum(xn * lnw_ref[:] + lnb_ref[:], 0.0)

  @pl.when(pl.program_id(0) == 0)
  def _():
    pool_ref[:] = jnp.zeros_like(pool_ref)

  pool_ref[:] += _segdot(pm, res)


def _head(s_ref, deg_ref, d1_ref, d2_ref, fwa_ref, fwb_ref, fwc_ref,
          fwd_ref, f1b_ref, f2w_ref, f2b_ref, ow_ref, ob_ref, out_ref):
  cnt = jnp.maximum(deg_ref[0, :], 1.0)
  s = s_ref[:]
  emb = s + s / cnt[:, None]
  e1 = emb[0:B]
  e2 = emb[B:2 * B]
  hh = (jnp.dot(e1, fwa_ref[:], preferred_element_type=jnp.float32,
                precision=HIGHEST)
        + jnp.dot(e2, fwb_ref[:], preferred_element_type=jnp.float32,
                  precision=HIGHEST)
        + jnp.dot(d1_ref[:], fwc_ref[:], preferred_element_type=jnp.float32,
                  precision=HIGHEST)
        + jnp.dot(d2_ref[:], fwd_ref[:], preferred_element_type=jnp.float32,
                  precision=HIGHEST)
        + f1b_ref[:])
  hh = jnp.maximum(hh, 0.0)
  hh = jnp.maximum(
      jnp.dot(hh, f2w_ref[:], preferred_element_type=jnp.float32,
              precision=HIGHEST) + f2b_ref[:], 0.0)
  out_ref[:] = (jnp.dot(hh, ow_ref[:], preferred_element_type=jnp.float32,
                        precision=HIGHEST) + ob_ref[:])


def _row_spec(width):
  return pl.BlockSpec((BLK, width), lambda i: (i, 0))


def _fix_spec(shape):
  return pl.BlockSpec(shape, lambda i: (0, 0))


def kernel(g1_x, g1_edge_index, g1_batch, g2_x, g2_edge_index, g2_batch,
           d1, d2,
           nn1_w1, nn1_b1, nn1_w2, nn1_b2, ln1_w, ln1_b,
           nn2_w1, nn2_b1, nn2_w2, nn2_b2, ln2_w, ln2_b,
           fc1_w, fc1_b, fc2_w, fc2_b, out_w, out_b):
  f32 = jnp.float32
  xs = jnp.concatenate([g1_x, g2_x], axis=0)
  srcf = jnp.stack([g1_edge_index[0], g2_edge_index[0] + N]).astype(jnp.int32)
  dstf = jnp.stack([g1_edge_index[1], g2_edge_index[1]]).astype(jnp.int32)

  def shaped(cfg):
    chunk, idxg, ngroup, _ = cfg
    shape = (NC, NS, ngroup, idxg, chunk)
    return srcf.reshape(shape), dstf.reshape(shape)

  src, dst = shaped(CFG128)
  src64, dst64 = shaped(CFG64)
  z128 = jnp.zeros((ROWS_PER_TILE, 128), f32)
  z64 = jnp.zeros((ROWS_PER_TILE, 64), f32)
  bat = jnp.concatenate([g1_batch, g2_batch + B]).astype(jnp.int32)
  bat = bat.reshape(2 * N, 1)

  agg1 = _edge_agg_128(xs, src, dst, z128)

  hm, s1a, segva, deg = pl.pallas_call(
      _mlp1_stats,
      grid=(NBLK,),
      in_specs=[
          _row_spec(128), _row_spec(128), _row_spec(1),
          _fix_spec((128, 64)), _fix_spec((1, 64)),
          _fix_spec((64, 64)), _fix_spec((1, 64)),
      ],
      out_specs=[
          _row_spec(64),
          _fix_spec((2 * B, 64)), _fix_spec((2 * B, 64)),
          _fix_spec((1, 2 * B)),
      ],
      out_shape=[
          jax.ShapeDtypeStruct((2 * N, 64), f32),
          jax.ShapeDtypeStruct((2 * B, 64), f32),
          jax.ShapeDtypeStruct((2 * B, 64), f32),
          jax.ShapeDtypeStruct((1, 2 * B), f32),
      ],
  )(xs, agg1, bat, nn1_w1, nn1_b1.reshape(1, -1), nn1_w2,
    nn1_b2.reshape(1, -1))

  h = pl.pallas_call(
      _ln1_apply,
      grid=(NBLK,),
      in_specs=[
          _row_spec(64), _row_spec(1),
          _fix_spec((2 * B, 64)), _fix_spec((2 * B, 64)),
          _fix_spec((1, 2 * B)), _fix_spec((1, 64)), _fix_spec((1, 64)),
      ],
      out_specs=_row_spec(64),
      out_shape=jax.ShapeDtypeStruct((2 * N, 64), f32),
  )(hm, bat, s1a, segva, deg, ln1_w.reshape(1, -1), ln1_b.reshape(1, -1))

  agg2 = _edge_agg_64(h, src64, dst64, z64)

  h2m, s1b, segvb = pl.pallas_call(
      _mlp2_stats,
      grid=(NBLK,),
      in_specs=[
          _row_spec(64), _row_spec(64), _row_spec(1),
          _fix_spec((64, 64)), _fix_spec((1, 64)),
          _fix_spec((64, 64)), _fix_spec((1, 64)),
      ],
      out_specs=[
          _row_spec(64),
          _fix_spec((2 * B, 64)), _fix_spec((2 * B, 64)),
      ],
      out_shape=[
          jax.ShapeDtypeStruct((2 * N, 64), f32),
          jax.ShapeDtypeStruct((2 * B, 64), f32),
          jax.ShapeDtypeStruct((2 * B, 64), f32),
      ],
  )(h, agg2, bat, nn2_w1, nn2_b1.reshape(1, -1), nn2_w2,
    nn2_b2.reshape(1, -1))

  pool = pl.pallas_call(
      _ln2_pool,
      grid=(NBLK,),
      in_specs=[
          _row_spec(64), _row_spec(1),
          _fix_spec((2 * B, 64)), _fix_spec((2 * B, 64)),
          _fix_spec((1, 2 * B)), _fix_spec((1, 64)), _fix_spec((1, 64)),
      ],
      out_specs=_fix_spec((2 * B, 64)),
      out_shape=jax.ShapeDtypeStruct((2 * B, 64), f32),
  )(h2m, bat, s1b, segvb, deg, ln2_w.reshape(1, -1), ln2_b.reshape(1, -1))

  out = pl.pallas_call(
      _head,
      out_shape=jax.ShapeDtypeStruct((B, 1), f32),
  )(pool, deg, d1, d2, fc1_w[0:B], fc1_w[B:2 * B], fc1_w[2 * B:2 * B + 5],
    fc1_w[2 * B + 5:2 * B + 10], fc1_b.reshape(1, -1), fc2_w,
    fc2_b.reshape(1, -1), out_w, out_b.reshape(1, -1))
  return out
